# baseline (speedup 1.0000x reference)
import functools

import jax
import jax.numpy as jnp
from jax import lax
from jax.experimental import pallas as pl
from jax.experimental.pallas import tpu as pltpu
from jax.experimental.pallas import tpu_sc as plsc

R = 128
BASIS_DIM = 9
DATA_DIM = 3 * BASIS_DIM + 1
N_STEPS = 128
STEP_SIZE = 0.01
BACKGROUND = 1.0
RGB_PADDING = 0.001

SH_C0 = 0.28209479177387814
SH_C1 = 0.4886025119029199
SH_C2 = (1.0925484305920792, -1.0925484305920792, 0.31539156525252005,
         -1.0925484305920792, 0.5462742152960396)

NC = 2
NS = 16
LANES = 16
NW = NC * NS
N_GROUPS = N_STEPS // LANES


def _sc_body(rays_per_w, grid_ref, rp_ref, out_ref, rp_v, idx_v, rows_v,
             outst_v, sem):
    wid = lax.axis_index("s") * NC + lax.axis_index("c")
    ray0 = wid * rays_per_w
    pltpu.sync_copy(rp_ref.at[pl.ds(ray0, rays_per_w)], rp_v)
    iota = lax.iota(jnp.int32, LANES)
    iota_f = iota.astype(jnp.float32)

    def ray_body(i, carry):
        params = rp_v[i]

        def bc(k):
            return params.at[jnp.full((LANES,), k, jnp.int32)].get(
                mode="promise_in_bounds")

        ox, oy, oz = bc(0), bc(1), bc(2)
        dx, dy, dz = bc(3), bc(4), bc(5)
        tmn, tmx = bc(6), bc(7)
        sh = [bc(8 + k) for k in range(BASIS_DIM - 1)]

        for g in range(N_GROUPS):
            t = tmn + STEP_SIZE * (iota_f + (g * LANES + 0.5))
            fx = jnp.clip((ox + t * dx) * R, 0.0, R - 1.0)
            fy = jnp.clip((oy + t * dy) * R, 0.0, R - 1.0)
            fz = jnp.clip((oz + t * dz) * R, 0.0, R - 1.0)
            lin = (fx.astype(jnp.int32) * R + fy.astype(jnp.int32)) * R \
                + fz.astype(jnp.int32)
            idx_v[pl.ds(g * LANES, LANES)] = lin
        pltpu.async_copy(grid_ref.at[idx_v], rows_v, sem).wait()

        for g in range(N_GROUPS):
            rows = g * LANES + iota

            def feat(f):
                return plsc.load_gather(
                    rows_v, [rows, jnp.full((LANES,), f, jnp.int32)])

            for c in range(3):
                acc = SH_C0 * feat(BASIS_DIM * c)
                for k in range(BASIS_DIM - 1):
                    acc = acc + sh[k] * feat(BASIS_DIM * c + k + 1)
                outst_v[c, pl.ds(g * LANES, LANES)] = acc
            t = tmn + STEP_SIZE * (iota_f + (g * LANES + 0.5))
            raw = jnp.where(t < tmx, feat(DATA_DIM - 1), -40.0)
            outst_v[3, pl.ds(g * LANES, LANES)] = raw
        pltpu.sync_copy(outst_v, out_ref.at[ray0 + i])
        return carry

    lax.fori_loop(0, rays_per_w, ray_body, 0)


@functools.lru_cache(maxsize=None)
def _make_sc_render(batch):
    rays_per_w = batch // NW
    return pl.kernel(
        functools.partial(_sc_body, rays_per_w),
        out_type=jax.ShapeDtypeStruct((batch, 4, N_STEPS), jnp.float32),
        mesh=plsc.VectorSubcoreMesh(core_axis_name="c", subcore_axis_name="s"),
        compiler_params=pltpu.CompilerParams(
            needs_layout_passes=False, use_tc_tiling_on_sc=False),
        scratch_types=[
            pltpu.VMEM((rays_per_w, 16), jnp.float32),
            pltpu.VMEM((N_STEPS,), jnp.int32),
            pltpu.VMEM((N_STEPS, DATA_DIM), jnp.float32),
            pltpu.VMEM((4, N_STEPS), jnp.float32),
            pltpu.SemaphoreType.DMA,
        ],
    )


def _composite_body(data_ref, tri_ref, o_ref):
    d = data_ref[...]
    raw = d[:, 3, :] - 1.0
    sig = jnp.maximum(raw, 0.0) + jnp.log(1.0 + jnp.exp(-jnp.abs(raw)))
    alpha = 1.0 - jnp.exp(-STEP_SIZE * sig)
    csum_ex = lax.dot_general(
        sig, tri_ref[...], (((1,), (0,)), ((), ())),
        precision=lax.Precision.HIGHEST, preferred_element_type=jnp.float32)
    w = jnp.exp(-STEP_SIZE * csum_ex) * alpha
    outs = []
    for c in range(3):
        rgb = (1.0 + 2.0 * RGB_PADDING) / (1.0 + jnp.exp(-d[:, c, :])) \
            - RGB_PADDING
        outs.append(jnp.sum(w * rgb, axis=1, keepdims=True))
    t_last = jnp.exp(-STEP_SIZE * jnp.sum(sig, axis=1, keepdims=True))
    o_ref[...] = jnp.concatenate(outs, axis=1) + t_last * BACKGROUND


def _composite(fused, interpret=False):
    batch = fused.shape[0]
    blk = 512
    tri = (jnp.arange(N_STEPS)[:, None] < jnp.arange(N_STEPS)[None, :]
           ).astype(jnp.float32)
    return pl.pallas_call(
        _composite_body,
        grid=(batch // blk,),
        in_specs=[
            pl.BlockSpec((blk, 4, N_STEPS), lambda i: (i, 0, 0)),
            pl.BlockSpec((N_STEPS, N_STEPS), lambda i: (0, 0)),
        ],
        out_specs=pl.BlockSpec((blk, 3), lambda i: (i, 0)),
        out_shape=jax.ShapeDtypeStruct((batch, 3), jnp.float32),
        interpret=interpret,
    )(fused, tri)


def _ray_params(origins, dirs, viewdirs):
    dirs = dirs / jnp.linalg.norm(dirs, axis=-1, keepdims=True)
    viewdirs = viewdirs / jnp.linalg.norm(viewdirs, axis=-1, keepdims=True)
    invdir = 1.0 / (dirs + 1e-9)
    t1 = -origins * invdir
    t2 = t1 + invdir
    tmin = jnp.maximum(jnp.max(jnp.minimum(t1, t2), axis=-1), 0.0)
    tmax = jnp.min(jnp.maximum(t1, t2), axis=-1)
    x, y, z = viewdirs[:, 0], viewdirs[:, 1], viewdirs[:, 2]
    sh = jnp.stack([
        -SH_C1 * y,
        SH_C1 * z,
        -SH_C1 * x,
        SH_C2[0] * x * y,
        SH_C2[1] * y * z,
        SH_C2[2] * (2.0 * z * z - x * x - y * y),
        SH_C2[3] * x * z,
        SH_C2[4] * (x * x - y * y),
    ], axis=-1)
    return jnp.concatenate(
        [origins, dirs, tmin[:, None], tmax[:, None], sh], axis=1)


def kernel(origins, dirs, viewdirs, grid):
    batch = origins.shape[0]
    rp = _ray_params(origins, dirs, viewdirs)
    grid_flat = grid.reshape(R * R * R, DATA_DIM)
    fused = _make_sc_render(batch)(grid_flat, rp)
    return _composite(fused)


# speedup vs baseline: 1.1164x; 1.1164x over previous
import functools

import jax
import jax.numpy as jnp
from jax import lax
from jax.experimental import pallas as pl
from jax.experimental.pallas import tpu as pltpu
from jax.experimental.pallas import tpu_sc as plsc

R = 128
BASIS_DIM = 9
DATA_DIM = 3 * BASIS_DIM + 1
ROW_PAD = 32
N_STEPS = 128
STEP_SIZE = 0.01
BACKGROUND = 1.0
RGB_PADDING = 0.001

SH_C0 = 0.28209479177387814
SH_C1 = 0.4886025119029199
SH_C2 = (1.0925484305920792, -1.0925484305920792, 0.31539156525252005,
         -1.0925484305920792, 0.5462742152960396)

NC = 2
NS = 16
LANES = 16
NW = NC * NS
N_GROUPS = N_STEPS // LANES
XS_PER_W = R // NW
YC = 16

_SC_PARAMS = pltpu.CompilerParams(
    needs_layout_passes=False, use_tc_tiling_on_sc=False)


def _sc_permute_body(g_ref, p_ref, in_v, out_v, sem):
    wid = lax.axis_index("s") * NC + lax.axis_index("c")
    iota = lax.iota(jnp.int32, LANES)

    def slab_body(si, carry0):
        x = wid * XS_PER_W + si

        def chunk_body(yc, carry1):
            y0 = yc * YC
            cps = [pltpu.async_copy(
                g_ref.at[pl.ds((x * DATA_DIM + c) * R + y0, YC)],
                in_v.at[c], sem) for c in range(DATA_DIM)]
            for cp in cps:
                cp.wait()

            def y_body(yl, carry2):
                for c in range(DATA_DIM):
                    colv = jnp.full((LANES,), c, jnp.int32)
                    for zg in range(R // LANES):
                        v = in_v[c, yl, pl.ds(zg * LANES, LANES)]
                        rows = yl * R + zg * LANES + iota
                        plsc.store_scatter(out_v, [rows, colv], v)
                return carry2

            lax.fori_loop(0, YC, y_body, 0)
            pltpu.sync_copy(out_v, p_ref.at[pl.ds((x * R + y0) * R, YC * R)])
            return carry1

        lax.fori_loop(0, R // YC, chunk_body, 0)
        return carry0

    lax.fori_loop(0, XS_PER_W, slab_body, 0)


_sc_permute = pl.kernel(
    _sc_permute_body,
    out_type=jax.ShapeDtypeStruct((R * R * R, ROW_PAD), jnp.float32),
    mesh=plsc.VectorSubcoreMesh(core_axis_name="c", subcore_axis_name="s"),
    compiler_params=_SC_PARAMS,
    scratch_types=[
        pltpu.VMEM((DATA_DIM, YC, R), jnp.float32),
        pltpu.VMEM((YC * R, ROW_PAD), jnp.float32),
        pltpu.SemaphoreType.DMA,
    ],
)


def _sc_render_body(rays_per_w, p_ref, rp_ref, out_ref, rp_v, idx_v, rows_v,
                    outst_v, sem):
    wid = lax.axis_index("s") * NC + lax.axis_index("c")
    ray0 = wid * rays_per_w
    pltpu.sync_copy(rp_ref.at[pl.ds(ray0, rays_per_w)], rp_v)
    iota = lax.iota(jnp.int32, LANES)
    iota_f = iota.astype(jnp.float32)

    def ray_body(i, carry):
        params = rp_v[i]

        def bc(k):
            return params.at[jnp.full((LANES,), k, jnp.int32)].get(
                mode="promise_in_bounds")

        ox, oy, oz = bc(0), bc(1), bc(2)
        dx, dy, dz = bc(3), bc(4), bc(5)
        tmn, tmx = bc(6), bc(7)
        sh = [bc(8 + k) for k in range(BASIS_DIM - 1)]

        for g in range(N_GROUPS):
            t = tmn + STEP_SIZE * (iota_f + (g * LANES + 0.5))
            fx = jnp.clip((ox + t * dx) * R, 0.0, R - 1.0)
            fy = jnp.clip((oy + t * dy) * R, 0.0, R - 1.0)
            fz = jnp.clip((oz + t * dz) * R, 0.0, R - 1.0)
            lin = (fx.astype(jnp.int32) * R + fy.astype(jnp.int32)) * R \
                + fz.astype(jnp.int32)
            idx_v[pl.ds(g * LANES, LANES)] = lin
        pltpu.async_copy(p_ref.at[idx_v], rows_v, sem).wait()

        for g in range(N_GROUPS):
            rows = g * LANES + iota

            def feat(f):
                return plsc.load_gather(
                    rows_v, [rows, jnp.full((LANES,), f, jnp.int32)])

            for c in range(3):
                acc = SH_C0 * feat(BASIS_DIM * c)
                for k in range(BASIS_DIM - 1):
                    acc = acc + sh[k] * feat(BASIS_DIM * c + k + 1)
                outst_v[c, pl.ds(g * LANES, LANES)] = acc
            t = tmn + STEP_SIZE * (iota_f + (g * LANES + 0.5))
            raw = jnp.where(t < tmx, feat(DATA_DIM - 1), -40.0)
            outst_v[3, pl.ds(g * LANES, LANES)] = raw
        pltpu.sync_copy(outst_v, out_ref.at[ray0 + i])
        return carry

    lax.fori_loop(0, rays_per_w, ray_body, 0)


@functools.lru_cache(maxsize=None)
def _make_sc_render(batch):
    rays_per_w = batch // NW
    return pl.kernel(
        functools.partial(_sc_render_body, rays_per_w),
        out_type=jax.ShapeDtypeStruct((batch, 4, N_STEPS), jnp.float32),
        mesh=plsc.VectorSubcoreMesh(core_axis_name="c", subcore_axis_name="s"),
        compiler_params=_SC_PARAMS,
        scratch_types=[
            pltpu.VMEM((rays_per_w, 16), jnp.float32),
            pltpu.VMEM((N_STEPS,), jnp.int32),
            pltpu.VMEM((N_STEPS, ROW_PAD), jnp.float32),
            pltpu.VMEM((4, N_STEPS), jnp.float32),
            pltpu.SemaphoreType.DMA,
        ],
    )


def _composite_body(data_ref, tri_ref, o_ref):
    d = data_ref[...]
    raw = d[:, 3, :] - 1.0
    sig = jnp.maximum(raw, 0.0) + jnp.log(1.0 + jnp.exp(-jnp.abs(raw)))
    alpha = 1.0 - jnp.exp(-STEP_SIZE * sig)
    csum_ex = lax.dot_general(
        sig, tri_ref[...], (((1,), (0,)), ((), ())),
        precision=lax.Precision.HIGHEST, preferred_element_type=jnp.float32)
    w = jnp.exp(-STEP_SIZE * csum_ex) * alpha
    outs = []
    for c in range(3):
        rgb = (1.0 + 2.0 * RGB_PADDING) / (1.0 + jnp.exp(-d[:, c, :])) \
            - RGB_PADDING
        outs.append(jnp.sum(w * rgb, axis=1, keepdims=True))
    t_last = jnp.exp(-STEP_SIZE * jnp.sum(sig, axis=1, keepdims=True))
    o_ref[...] = jnp.concatenate(outs, axis=1) + t_last * BACKGROUND


def _composite(fused, interpret=False):
    batch = fused.shape[0]
    blk = 512
    tri = (jnp.arange(N_STEPS)[:, None] < jnp.arange(N_STEPS)[None, :]
           ).astype(jnp.float32)
    return pl.pallas_call(
        _composite_body,
        grid=(batch // blk,),
        in_specs=[
            pl.BlockSpec((blk, 4, N_STEPS), lambda i: (i, 0, 0)),
            pl.BlockSpec((N_STEPS, N_STEPS), lambda i: (0, 0)),
        ],
        out_specs=pl.BlockSpec((blk, 3), lambda i: (i, 0)),
        out_shape=jax.ShapeDtypeStruct((batch, 3), jnp.float32),
        interpret=interpret,
    )(fused, tri)


def _ray_params(origins, dirs, viewdirs):
    dirs = dirs / jnp.linalg.norm(dirs, axis=-1, keepdims=True)
    viewdirs = viewdirs / jnp.linalg.norm(viewdirs, axis=-1, keepdims=True)
    invdir = 1.0 / (dirs + 1e-9)
    t1 = -origins * invdir
    t2 = t1 + invdir
    tmin = jnp.maximum(jnp.max(jnp.minimum(t1, t2), axis=-1), 0.0)
    tmax = jnp.min(jnp.maximum(t1, t2), axis=-1)
    x, y, z = viewdirs[:, 0], viewdirs[:, 1], viewdirs[:, 2]
    sh = jnp.stack([
        -SH_C1 * y,
        SH_C1 * z,
        -SH_C1 * x,
        SH_C2[0] * x * y,
        SH_C2[1] * y * z,
        SH_C2[2] * (2.0 * z * z - x * x - y * y),
        SH_C2[3] * x * z,
        SH_C2[4] * (x * x - y * y),
    ], axis=-1)
    return jnp.concatenate(
        [origins, dirs, tmin[:, None], tmax[:, None], sh], axis=1)


def kernel(origins, dirs, viewdirs, grid):
    batch = origins.shape[0]
    rp = _ray_params(origins, dirs, viewdirs)
    g_rows = grid.transpose(0, 3, 1, 2).reshape(R * DATA_DIM * R, R)
    table = _sc_permute(g_rows)
    fused = _make_sc_render(batch)(table, rp)
    return _composite(fused)


# speedup vs baseline: 1.4242x; 1.2757x over previous
import functools

import jax
import jax.numpy as jnp
from jax import lax
from jax.experimental import pallas as pl
from jax.experimental.pallas import tpu as pltpu
from jax.experimental.pallas import tpu_sc as plsc

R = 128
BASIS_DIM = 9
DATA_DIM = 3 * BASIS_DIM + 1
ROW_PAD = 32
N_STEPS = 128
STEP_SIZE = 0.01
BACKGROUND = 1.0
RGB_PADDING = 0.001

SH_C0 = 0.28209479177387814
SH_C1 = 0.4886025119029199
SH_C2 = (1.0925484305920792, -1.0925484305920792, 0.31539156525252005,
         -1.0925484305920792, 0.5462742152960396)

NC = 2
NS = 16
LANES = 16
NW = NC * NS
N_GROUPS = N_STEPS // LANES
XS_PER_W = R // NW
YC = 8
OUT_PAD = 33

_SC_PARAMS = pltpu.CompilerParams(
    needs_layout_passes=False, use_tc_tiling_on_sc=False)


def _sc_permute_body(g_ref, p_ref, in_v, out_v, sem):
    wid = lax.axis_index("s") * NC + lax.axis_index("c")
    iota = lax.iota(jnp.int32, LANES)

    def slab_body(si, carry0):
        x = wid * XS_PER_W + si

        def chunk_body(yc, carry1):
            y0 = yc * YC
            cps = [pltpu.async_copy(
                g_ref.at[pl.ds((x * DATA_DIM + c) * R + y0, YC)],
                in_v.at[c], sem) for c in range(DATA_DIM)]
            for cp in cps:
                cp.wait()

            def y_body(yl, carry2):
                for c in range(DATA_DIM):
                    colv = jnp.full((LANES,), c, jnp.int32)
                    for zg in range(R // LANES):
                        v = in_v[c, yl, pl.ds(zg * LANES, LANES)]
                        rows = yl * R + zg * LANES + iota
                        plsc.store_scatter(out_v, [rows, colv], v)
                return carry2

            lax.fori_loop(0, YC, y_body, 0)
            pltpu.sync_copy(out_v.at[:, pl.ds(0, ROW_PAD)],
                            p_ref.at[pl.ds((x * R + y0) * R, YC * R)])
            return carry1

        lax.fori_loop(0, R // YC, chunk_body, 0)
        return carry0

    lax.fori_loop(0, XS_PER_W, slab_body, 0)


_sc_permute = pl.kernel(
    _sc_permute_body,
    out_type=jax.ShapeDtypeStruct((R * R * R, ROW_PAD), jnp.float32),
    mesh=plsc.VectorSubcoreMesh(core_axis_name="c", subcore_axis_name="s"),
    compiler_params=_SC_PARAMS,
    scratch_types=[
        pltpu.VMEM((DATA_DIM, YC, R), jnp.float32),
        pltpu.VMEM((YC * R, OUT_PAD), jnp.float32),
        pltpu.SemaphoreType.DMA,
    ],
)


def _sc_render_body(rays_per_w, p_ref, rp_ref, out_ref, rp_v, idx_v, rows_v,
                    outst_v, sem):
    wid = lax.axis_index("s") * NC + lax.axis_index("c")
    ray0 = wid * rays_per_w
    pltpu.sync_copy(rp_ref.at[pl.ds(ray0, rays_per_w)], rp_v)
    iota = lax.iota(jnp.int32, LANES)
    iota_f = iota.astype(jnp.float32)

    def ray_body(i, carry):
        params = rp_v[i]

        def bc(k):
            return params.at[jnp.full((LANES,), k, jnp.int32)].get(
                mode="promise_in_bounds")

        ox, oy, oz = bc(0), bc(1), bc(2)
        dx, dy, dz = bc(3), bc(4), bc(5)
        tmn, tmx = bc(6), bc(7)
        sh = [bc(8 + k) for k in range(BASIS_DIM - 1)]

        for g in range(N_GROUPS):
            t = tmn + STEP_SIZE * (iota_f + (g * LANES + 0.5))
            fx = jnp.clip((ox + t * dx) * R, 0.0, R - 1.0)
            fy = jnp.clip((oy + t * dy) * R, 0.0, R - 1.0)
            fz = jnp.clip((oz + t * dz) * R, 0.0, R - 1.0)
            lin = (fx.astype(jnp.int32) * R + fy.astype(jnp.int32)) * R \
                + fz.astype(jnp.int32)
            idx_v[pl.ds(g * LANES, LANES)] = lin
        pltpu.async_copy(p_ref.at[idx_v], rows_v, sem).wait()

        for g in range(N_GROUPS):
            rows = g * LANES + iota

            def feat(f):
                return plsc.load_gather(
                    rows_v, [rows, jnp.full((LANES,), f, jnp.int32)])

            for c in range(3):
                acc = SH_C0 * feat(BASIS_DIM * c)
                for k in range(BASIS_DIM - 1):
                    acc = acc + sh[k] * feat(BASIS_DIM * c + k + 1)
                outst_v[c, pl.ds(g * LANES, LANES)] = acc
            t = tmn + STEP_SIZE * (iota_f + (g * LANES + 0.5))
            raw = jnp.where(t < tmx, feat(DATA_DIM - 1), -40.0)
            outst_v[3, pl.ds(g * LANES, LANES)] = raw
        pltpu.sync_copy(outst_v, out_ref.at[ray0 + i])
        return carry

    lax.fori_loop(0, rays_per_w, ray_body, 0)


@functools.lru_cache(maxsize=None)
def _make_sc_render(batch):
    rays_per_w = batch // NW
    return pl.kernel(
        functools.partial(_sc_render_body, rays_per_w),
        out_type=jax.ShapeDtypeStruct((batch, 4, N_STEPS), jnp.float32),
        mesh=plsc.VectorSubcoreMesh(core_axis_name="c", subcore_axis_name="s"),
        compiler_params=_SC_PARAMS,
        scratch_types=[
            pltpu.VMEM((rays_per_w, 16), jnp.float32),
            pltpu.VMEM((N_STEPS,), jnp.int32),
            pltpu.VMEM((N_STEPS, ROW_PAD), jnp.float32),
            pltpu.VMEM((4, N_STEPS), jnp.float32),
            pltpu.SemaphoreType.DMA,
        ],
    )


def _composite_body(data_ref, tri_ref, o_ref):
    d = data_ref[...]
    raw = d[:, 3, :] - 1.0
    sig = jnp.maximum(raw, 0.0) + jnp.log(1.0 + jnp.exp(-jnp.abs(raw)))
    alpha = 1.0 - jnp.exp(-STEP_SIZE * sig)
    csum_ex = lax.dot_general(
        sig, tri_ref[...], (((1,), (0,)), ((), ())),
        precision=lax.Precision.HIGHEST, preferred_element_type=jnp.float32)
    w = jnp.exp(-STEP_SIZE * csum_ex) * alpha
    outs = []
    for c in range(3):
        rgb = (1.0 + 2.0 * RGB_PADDING) / (1.0 + jnp.exp(-d[:, c, :])) \
            - RGB_PADDING
        outs.append(jnp.sum(w * rgb, axis=1, keepdims=True))
    t_last = jnp.exp(-STEP_SIZE * jnp.sum(sig, axis=1, keepdims=True))
    o_ref[...] = jnp.concatenate(outs, axis=1) + t_last * BACKGROUND


def _composite(fused, interpret=False):
    batch = fused.shape[0]
    blk = 512
    tri = (jnp.arange(N_STEPS)[:, None] < jnp.arange(N_STEPS)[None, :]
           ).astype(jnp.float32)
    return pl.pallas_call(
        _composite_body,
        grid=(batch // blk,),
        in_specs=[
            pl.BlockSpec((blk, 4, N_STEPS), lambda i: (i, 0, 0)),
            pl.BlockSpec((N_STEPS, N_STEPS), lambda i: (0, 0)),
        ],
        out_specs=pl.BlockSpec((blk, 3), lambda i: (i, 0)),
        out_shape=jax.ShapeDtypeStruct((batch, 3), jnp.float32),
        interpret=interpret,
    )(fused, tri)


def _ray_params(origins, dirs, viewdirs):
    dirs = dirs / jnp.linalg.norm(dirs, axis=-1, keepdims=True)
    viewdirs = viewdirs / jnp.linalg.norm(viewdirs, axis=-1, keepdims=True)
    invdir = 1.0 / (dirs + 1e-9)
    t1 = -origins * invdir
    t2 = t1 + invdir
    tmin = jnp.maximum(jnp.max(jnp.minimum(t1, t2), axis=-1), 0.0)
    tmax = jnp.min(jnp.maximum(t1, t2), axis=-1)
    x, y, z = viewdirs[:, 0], viewdirs[:, 1], viewdirs[:, 2]
    sh = jnp.stack([
        -SH_C1 * y,
        SH_C1 * z,
        -SH_C1 * x,
        SH_C2[0] * x * y,
        SH_C2[1] * y * z,
        SH_C2[2] * (2.0 * z * z - x * x - y * y),
        SH_C2[3] * x * z,
        SH_C2[4] * (x * x - y * y),
    ], axis=-1)
    return jnp.concatenate(
        [origins, dirs, tmin[:, None], tmax[:, None], sh], axis=1)


def kernel(origins, dirs, viewdirs, grid):
    batch = origins.shape[0]
    rp = _ray_params(origins, dirs, viewdirs)
    g_rows = grid.transpose(0, 3, 1, 2).reshape(R * DATA_DIM * R, R)
    table = _sc_permute(g_rows)
    fused = _make_sc_render(batch)(table, rp)
    return _composite(fused)


# speedup vs baseline: 3.1829x; 2.2348x over previous
import functools

import jax
import jax.numpy as jnp
from jax import lax
from jax.experimental import pallas as pl
from jax.experimental.pallas import tpu as pltpu
from jax.experimental.pallas import tpu_sc as plsc

R = 128
BASIS_DIM = 9
DATA_DIM = 3 * BASIS_DIM + 1
N_STEPS = 128
STEP_SIZE = 0.01
BACKGROUND = 1.0
RGB_PADDING = 0.001

SH_C0 = 0.28209479177387814
SH_C1 = 0.4886025119029199
SH_C2 = (1.0925484305920792, -1.0925484305920792, 0.31539156525252005,
         -1.0925484305920792, 0.5462742152960396)

NC = 2
NS = 16
LANES = 16
NW = NC * NS
N_GROUPS = N_STEPS // LANES
XS_PER_W = R // NW
YC = 8
NCHUNK = R // YC
ROW_W = 16
STG_W = 17
TR_W = 129
NPAIR = 13
OUT_BATCH = 16

_SC_PARAMS = pltpu.CompilerParams(
    needs_layout_passes=False, use_tc_tiling_on_sc=False)

_HI = 0xFFFF0000


def _pack_words(vecs):
    u = [plsc.bitcast(v, jnp.uint32) for v in vecs]
    words = []
    for p in range(NPAIR):
        w = lax.shift_right_logical(u[2 * p], jnp.uint32(16)) | (u[2 * p + 1] & jnp.uint32(_HI))
        words.append(plsc.bitcast(w, jnp.int32))
    words.append(plsc.bitcast(u[26], jnp.int32))
    words.append(plsc.bitcast(u[27], jnp.int32))
    return words


def _sc_permute_body(g_ref, p_ref, in0, in1, out_v, sem0, sem1):
    wid = lax.axis_index("s") * NC + lax.axis_index("c")
    iota = lax.iota(jnp.int32, LANES)
    total = XS_PER_W * NCHUNK

    def fire(ci, buf, sem):
        ci = jnp.minimum(ci, total - 1)
        x = wid * XS_PER_W + ci // NCHUNK
        y0 = (ci % NCHUNK) * YC
        for c in range(DATA_DIM):
            pltpu.async_copy(
                g_ref.at[pl.ds((x * DATA_DIM + c) * R + y0, YC)],
                buf.at[c], sem)

    def drain(buf, sem):
        for c in range(DATA_DIM):
            pltpu.make_async_copy(g_ref.at[pl.ds(0, YC)], buf.at[c],
                                  sem).wait()

    def run(ci, buf):
        def y_body(yl, carry):
            for zg in range(R // LANES):
                vecs = [buf[c, yl, pl.ds(zg * LANES, LANES)]
                        for c in range(DATA_DIM)]
                words = _pack_words(vecs)
                rows = yl * R + zg * LANES + iota
                for col in range(15):
                    plsc.store_scatter(
                        out_v, [rows, jnp.full((LANES,), col, jnp.int32)],
                        words[col])
            return carry

        lax.fori_loop(0, YC, y_body, 0)
        x = wid * XS_PER_W + ci // NCHUNK
        y0 = (ci % NCHUNK) * YC
        pltpu.sync_copy(out_v.at[:, pl.ds(0, ROW_W)],
                        p_ref.at[pl.ds((x * R + y0) * R, YC * R)])

    fire(0, in0, sem0)

    def pair_body(g, carry):
        c0 = 2 * g
        fire(c0 + 1, in1, sem1)
        drain(in0, sem0)
        run(c0, in0)
        fire(c0 + 2, in0, sem0)
        drain(in1, sem1)
        run(c0 + 1, in1)
        return carry

    lax.fori_loop(0, total // 2, pair_body, 0)
    drain(in0, sem0)


_sc_permute = pl.kernel(
    _sc_permute_body,
    out_type=jax.ShapeDtypeStruct((R * R * R, ROW_W), jnp.int32),
    mesh=plsc.VectorSubcoreMesh(core_axis_name="c", subcore_axis_name="s"),
    compiler_params=_SC_PARAMS,
    scratch_types=[
        pltpu.VMEM((DATA_DIM, YC, R), jnp.float32),
        pltpu.VMEM((DATA_DIM, YC, R), jnp.float32),
        pltpu.VMEM((YC * R, STG_W), jnp.int32),
        pltpu.SemaphoreType.DMA,
        pltpu.SemaphoreType.DMA,
    ],
)


def _sc_render_body(rays_per_w, p_ref, rp_ref, out_ref, rp_v, idx0, idx1,
                    rows0, rows1, tr_v, outst_v, sem0, sem1):
    wid = lax.axis_index("s") * NC + lax.axis_index("c")
    ray0 = wid * rays_per_w
    pltpu.sync_copy(rp_ref.at[pl.ds(ray0, rays_per_w)], rp_v)
    iota = lax.iota(jnp.int32, LANES)
    iota_f = iota.astype(jnp.float32)

    def ray_ctx(i):
        params = rp_v[jnp.minimum(i, rays_per_w - 1)]

        def bc(k):
            return params.at[jnp.full((LANES,), k, jnp.int32)].get(
                mode="promise_in_bounds")
        return [bc(k) for k in range(16)]

    def fill_idx(ctx, idx_v):
        ox, oy, oz, dx, dy, dz, tmn = ctx[:7]
        for g in range(N_GROUPS):
            t = tmn + STEP_SIZE * (iota_f + (g * LANES + 0.5))
            fx = jnp.clip((ox + t * dx) * R, 0.0, R - 1.0)
            fy = jnp.clip((oy + t * dy) * R, 0.0, R - 1.0)
            fz = jnp.clip((oz + t * dz) * R, 0.0, R - 1.0)
            lin = (fx.astype(jnp.int32) * R + fy.astype(jnp.int32)) * R \
                + fz.astype(jnp.int32)
            idx_v[pl.ds(g * LANES, LANES)] = lin

    def fire(idx_v, rows_v, sem):
        pltpu.async_copy(p_ref.at[idx_v], rows_v, sem)

    def drain(idx_v, rows_v, sem):
        pltpu.make_async_copy(p_ref.at[idx_v], rows_v, sem).wait()

    def run(i, ctx, rows_v):
        tmn, tmx = ctx[6], ctx[7]
        sh = ctx[8:16]
        slot = lax.rem(i, OUT_BATCH)
        for s in range(N_STEPS):
            plsc.store_scatter(
                tr_v, [iota, jnp.full((LANES,), s, jnp.int32)], rows_v[s])
        for g in range(N_GROUPS):
            def word(w):
                return tr_v[w, pl.ds(g * LANES, LANES)]

            wv = [word(w) for w in range(15)]
            uv = [plsc.bitcast(w, jnp.uint32) for w in wv[:NPAIR]]
            fe = [plsc.bitcast(lax.shift_left(u, jnp.uint32(16)), jnp.float32)
                  for u in uv]
            fo = [plsc.bitcast(u & jnp.uint32(_HI), jnp.float32) for u in uv]

            def feat(f):
                if f == 26:
                    return plsc.bitcast(wv[13], jnp.float32)
                return fe[f // 2] if f % 2 == 0 else fo[f // 2]

            for c in range(3):
                acc = SH_C0 * feat(BASIS_DIM * c)
                for k in range(BASIS_DIM - 1):
                    acc = acc + sh[k] * feat(BASIS_DIM * c + k + 1)
                outst_v[slot, c, pl.ds(g * LANES, LANES)] = acc
            t = tmn + STEP_SIZE * (iota_f + (g * LANES + 0.5))
            raw = jnp.where(t < tmx, plsc.bitcast(wv[14], jnp.float32),
                            -40.0)
            outst_v[slot, 3, pl.ds(g * LANES, LANES)] = raw

    ctx0 = ray_ctx(0)
    fill_idx(ctx0, idx0)
    fire(idx0, rows0, sem0)

    def pair_body(g, carry):
        i0 = 2 * g
        c0 = ray_ctx(i0)
        c1 = ray_ctx(i0 + 1)
        fill_idx(c1, idx1)
        fire(idx1, rows1, sem1)
        drain(idx0, rows0, sem0)
        run(i0, c0, rows0)
        c2 = ray_ctx(i0 + 2)
        fill_idx(c2, idx0)
        fire(idx0, rows0, sem0)
        drain(idx1, rows1, sem1)
        run(i0 + 1, c1, rows1)

        @pl.when(lax.rem(i0 + 1, OUT_BATCH) == OUT_BATCH - 1)
        def _flush():
            pltpu.sync_copy(
                outst_v,
                out_ref.at[pl.ds(ray0 + (i0 + 1 - (OUT_BATCH - 1)),
                                 OUT_BATCH)])
        return carry

    lax.fori_loop(0, rays_per_w // 2, pair_body, 0)
    drain(idx0, rows0, sem0)


@functools.lru_cache(maxsize=None)
def _make_sc_render(batch):
    rays_per_w = batch // NW
    return pl.kernel(
        functools.partial(_sc_render_body, rays_per_w),
        out_type=jax.ShapeDtypeStruct((batch, 4, N_STEPS), jnp.float32),
        mesh=plsc.VectorSubcoreMesh(core_axis_name="c", subcore_axis_name="s"),
        compiler_params=_SC_PARAMS,
        scratch_types=[
            pltpu.VMEM((rays_per_w, 16), jnp.float32),
            pltpu.VMEM((N_STEPS,), jnp.int32),
            pltpu.VMEM((N_STEPS,), jnp.int32),
            pltpu.VMEM((N_STEPS, ROW_W), jnp.int32),
            pltpu.VMEM((N_STEPS, ROW_W), jnp.int32),
            pltpu.VMEM((LANES, TR_W), jnp.int32),
            pltpu.VMEM((OUT_BATCH, 4, N_STEPS), jnp.float32),
            pltpu.SemaphoreType.DMA,
            pltpu.SemaphoreType.DMA,
        ],
    )


def _composite_body(data_ref, tri_ref, o_ref):
    d = data_ref[...]
    raw = d[:, 3, :] - 1.0
    sig = jnp.maximum(raw, 0.0) + jnp.log(1.0 + jnp.exp(-jnp.abs(raw)))
    alpha = 1.0 - jnp.exp(-STEP_SIZE * sig)
    csum_ex = lax.dot_general(
        sig, tri_ref[...], (((1,), (0,)), ((), ())),
        precision=lax.Precision.HIGHEST, preferred_element_type=jnp.float32)
    w = jnp.exp(-STEP_SIZE * csum_ex) * alpha
    outs = []
    for c in range(3):
        rgb = (1.0 + 2.0 * RGB_PADDING) / (1.0 + jnp.exp(-d[:, c, :])) \
            - RGB_PADDING
        outs.append(jnp.sum(w * rgb, axis=1, keepdims=True))
    t_last = jnp.exp(-STEP_SIZE * jnp.sum(sig, axis=1, keepdims=True))
    o_ref[...] = jnp.concatenate(outs, axis=1) + t_last * BACKGROUND


def _composite(fused, interpret=False):
    batch = fused.shape[0]
    blk = 512
    tri = (jnp.arange(N_STEPS)[:, None] < jnp.arange(N_STEPS)[None, :]
           ).astype(jnp.float32)
    return pl.pallas_call(
        _composite_body,
        grid=(batch // blk,),
        in_specs=[
            pl.BlockSpec((blk, 4, N_STEPS), lambda i: (i, 0, 0)),
            pl.BlockSpec((N_STEPS, N_STEPS), lambda i: (0, 0)),
        ],
        out_specs=pl.BlockSpec((blk, 3), lambda i: (i, 0)),
        out_shape=jax.ShapeDtypeStruct((batch, 3), jnp.float32),
        interpret=interpret,
    )(fused, tri)


def _ray_params(origins, dirs, viewdirs):
    dirs = dirs / jnp.linalg.norm(dirs, axis=-1, keepdims=True)
    viewdirs = viewdirs / jnp.linalg.norm(viewdirs, axis=-1, keepdims=True)
    invdir = 1.0 / (dirs + 1e-9)
    t1 = -origins * invdir
    t2 = t1 + invdir
    tmin = jnp.maximum(jnp.max(jnp.minimum(t1, t2), axis=-1), 0.0)
    tmax = jnp.min(jnp.maximum(t1, t2), axis=-1)
    x, y, z = viewdirs[:, 0], viewdirs[:, 1], viewdirs[:, 2]
    sh = jnp.stack([
        -SH_C1 * y,
        SH_C1 * z,
        -SH_C1 * x,
        SH_C2[0] * x * y,
        SH_C2[1] * y * z,
        SH_C2[2] * (2.0 * z * z - x * x - y * y),
        SH_C2[3] * x * z,
        SH_C2[4] * (x * x - y * y),
    ], axis=-1)
    return jnp.concatenate(
        [origins, dirs, tmin[:, None], tmax[:, None], sh], axis=1)


def kernel(origins, dirs, viewdirs, grid):
    batch = origins.shape[0]
    rp = _ray_params(origins, dirs, viewdirs)
    g_rows = grid.transpose(0, 3, 1, 2).reshape(R * DATA_DIM * R, R)
    table = _sc_permute(g_rows)
    fused = _make_sc_render(batch)(table, rp)
    return _composite(fused)


# speedup vs baseline: 3.1973x; 1.0045x over previous
import functools

import jax
import jax.numpy as jnp
from jax import lax
from jax.experimental import pallas as pl
from jax.experimental.pallas import tpu as pltpu
from jax.experimental.pallas import tpu_sc as plsc

R = 128
BASIS_DIM = 9
DATA_DIM = 3 * BASIS_DIM + 1
N_STEPS = 128
STEP_SIZE = 0.01
BACKGROUND = 1.0
RGB_PADDING = 0.001

SH_C0 = 0.28209479177387814
SH_C1 = 0.4886025119029199
SH_C2 = (1.0925484305920792, -1.0925484305920792, 0.31539156525252005,
         -1.0925484305920792, 0.5462742152960396)

NC = 2
NS = 16
LANES = 16
NW = NC * NS
N_GROUPS = N_STEPS // LANES
XS_PER_W = R // NW
YC = 8
NCHUNK = R // YC
ROW_W = 16
STG_W = 17
TR_W = 129
NPAIR = 13
OUT_BATCH = 16

_SC_PARAMS = pltpu.CompilerParams(
    needs_layout_passes=False, use_tc_tiling_on_sc=False)

_HI = 0xFFFF0000


def _pack_words(vecs):
    u = [plsc.bitcast(v, jnp.uint32) for v in vecs]
    words = []
    for p in range(NPAIR):
        w = lax.shift_right_logical(u[2 * p], jnp.uint32(16)) | (u[2 * p + 1] & jnp.uint32(_HI))
        words.append(plsc.bitcast(w, jnp.int32))
    words.append(plsc.bitcast(u[26], jnp.int32))
    words.append(plsc.bitcast(u[27], jnp.int32))
    return words


def _sc_permute_body(g_ref, p_ref, in0, in1, out0, out1, sem0, sem1,
                     semo0, semo1):
    wid = lax.axis_index("s") * NC + lax.axis_index("c")
    iota = lax.iota(jnp.int32, LANES)
    total = XS_PER_W * NCHUNK

    def fire(ci, buf, sem):
        ci = jnp.minimum(ci, total - 1)
        x = wid * XS_PER_W + ci // NCHUNK
        y0 = (ci % NCHUNK) * YC
        for c in range(DATA_DIM):
            pltpu.async_copy(
                g_ref.at[pl.ds((x * DATA_DIM + c) * R + y0, YC)],
                buf.at[c], sem)

    def drain(buf, sem):
        for c in range(DATA_DIM):
            pltpu.make_async_copy(g_ref.at[pl.ds(0, YC)], buf.at[c],
                                  sem).wait()

    def drain_out(out_v, semo):
        pltpu.make_async_copy(
            out_v.at[:, pl.ds(0, ROW_W)],
            p_ref.at[pl.ds(0, YC * R)], semo).wait()

    def run(ci, buf, out_v, semo):
        @pl.when(ci >= 2)
        def _wait_prev():
            drain_out(out_v, semo)

        def y_body(yl, carry):
            for zg in range(R // LANES):
                vecs = [buf[c, yl, pl.ds(zg * LANES, LANES)]
                        for c in range(DATA_DIM)]
                words = _pack_words(vecs)
                rows = yl * R + zg * LANES + iota
                for col in range(15):
                    plsc.store_scatter(
                        out_v, [rows, jnp.full((LANES,), col, jnp.int32)],
                        words[col])
            return carry

        lax.fori_loop(0, YC, y_body, 0)
        x = wid * XS_PER_W + ci // NCHUNK
        y0 = (ci % NCHUNK) * YC
        pltpu.async_copy(out_v.at[:, pl.ds(0, ROW_W)],
                         p_ref.at[pl.ds((x * R + y0) * R, YC * R)], semo)

    fire(0, in0, sem0)

    def pair_body(g, carry):
        c0 = 2 * g
        fire(c0 + 1, in1, sem1)
        drain(in0, sem0)
        run(c0, in0, out0, semo0)
        fire(c0 + 2, in0, sem0)
        drain(in1, sem1)
        run(c0 + 1, in1, out1, semo1)
        return carry

    lax.fori_loop(0, total // 2, pair_body, 0)
    drain(in0, sem0)
    drain_out(out0, semo0)
    drain_out(out1, semo1)


_sc_permute = pl.kernel(
    _sc_permute_body,
    out_type=jax.ShapeDtypeStruct((R * R * R, ROW_W), jnp.int32),
    mesh=plsc.VectorSubcoreMesh(core_axis_name="c", subcore_axis_name="s"),
    compiler_params=_SC_PARAMS,
    scratch_types=[
        pltpu.VMEM((DATA_DIM, YC, R), jnp.float32),
        pltpu.VMEM((DATA_DIM, YC, R), jnp.float32),
        pltpu.VMEM((YC * R, STG_W), jnp.int32),
        pltpu.VMEM((YC * R, STG_W), jnp.int32),
        pltpu.SemaphoreType.DMA,
        pltpu.SemaphoreType.DMA,
        pltpu.SemaphoreType.DMA,
        pltpu.SemaphoreType.DMA,
    ],
)


def _sc_render_body(rays_per_w, p_ref, rp_ref, out_ref, rp_v, idx0, idx1,
                    rows0, rows1, tr_v, outst_v, sem0, sem1):
    wid = lax.axis_index("s") * NC + lax.axis_index("c")
    ray0 = wid * rays_per_w
    pltpu.sync_copy(rp_ref.at[pl.ds(ray0, rays_per_w)], rp_v)
    iota = lax.iota(jnp.int32, LANES)
    iota_f = iota.astype(jnp.float32)

    def ray_ctx(i):
        params = rp_v[jnp.minimum(i, rays_per_w - 1)]

        def bc(k):
            return params.at[jnp.full((LANES,), k, jnp.int32)].get(
                mode="promise_in_bounds")
        return [bc(k) for k in range(16)]

    def fill_idx(ctx, idx_v):
        ox, oy, oz, dx, dy, dz, tmn = ctx[:7]
        for g in range(N_GROUPS):
            t = tmn + STEP_SIZE * (iota_f + (g * LANES + 0.5))
            fx = jnp.clip((ox + t * dx) * R, 0.0, R - 1.0)
            fy = jnp.clip((oy + t * dy) * R, 0.0, R - 1.0)
            fz = jnp.clip((oz + t * dz) * R, 0.0, R - 1.0)
            lin = (fx.astype(jnp.int32) * R + fy.astype(jnp.int32)) * R \
                + fz.astype(jnp.int32)
            idx_v[pl.ds(g * LANES, LANES)] = lin

    def fire(idx_v, rows_v, sem):
        pltpu.async_copy(p_ref.at[idx_v], rows_v, sem)

    def drain(idx_v, rows_v, sem):
        pltpu.make_async_copy(p_ref.at[idx_v], rows_v, sem).wait()

    def run(i, ctx, rows_v):
        tmn, tmx = ctx[6], ctx[7]
        sh = ctx[8:16]
        slot = lax.rem(i, OUT_BATCH)
        for s in range(N_STEPS):
            plsc.store_scatter(
                tr_v, [iota, jnp.full((LANES,), s, jnp.int32)], rows_v[s])
        for g in range(N_GROUPS):
            def word(w):
                return tr_v[w, pl.ds(g * LANES, LANES)]

            wv = [word(w) for w in range(15)]
            uv = [plsc.bitcast(w, jnp.uint32) for w in wv[:NPAIR]]
            fe = [plsc.bitcast(lax.shift_left(u, jnp.uint32(16)), jnp.float32)
                  for u in uv]
            fo = [plsc.bitcast(u & jnp.uint32(_HI), jnp.float32) for u in uv]

            def feat(f):
                if f == 26:
                    return plsc.bitcast(wv[13], jnp.float32)
                return fe[f // 2] if f % 2 == 0 else fo[f // 2]

            for c in range(3):
                acc = SH_C0 * feat(BASIS_DIM * c)
                for k in range(BASIS_DIM - 1):
                    acc = acc + sh[k] * feat(BASIS_DIM * c + k + 1)
                outst_v[slot, c, pl.ds(g * LANES, LANES)] = acc
            t = tmn + STEP_SIZE * (iota_f + (g * LANES + 0.5))
            raw = jnp.where(t < tmx, plsc.bitcast(wv[14], jnp.float32),
                            -40.0)
            outst_v[slot, 3, pl.ds(g * LANES, LANES)] = raw

    ctx0 = ray_ctx(0)
    fill_idx(ctx0, idx0)
    fire(idx0, rows0, sem0)

    def pair_body(g, carry):
        i0 = 2 * g
        c0 = ray_ctx(i0)
        c1 = ray_ctx(i0 + 1)
        fill_idx(c1, idx1)
        fire(idx1, rows1, sem1)
        drain(idx0, rows0, sem0)
        run(i0, c0, rows0)
        c2 = ray_ctx(i0 + 2)
        fill_idx(c2, idx0)
        fire(idx0, rows0, sem0)
        drain(idx1, rows1, sem1)
        run(i0 + 1, c1, rows1)

        @pl.when(lax.rem(i0 + 1, OUT_BATCH) == OUT_BATCH - 1)
        def _flush():
            pltpu.sync_copy(
                outst_v,
                out_ref.at[pl.ds(ray0 + (i0 + 1 - (OUT_BATCH - 1)),
                                 OUT_BATCH)])
        return carry

    lax.fori_loop(0, rays_per_w // 2, pair_body, 0)
    drain(idx0, rows0, sem0)


@functools.lru_cache(maxsize=None)
def _make_sc_render(batch):
    rays_per_w = batch // NW
    return pl.kernel(
        functools.partial(_sc_render_body, rays_per_w),
        out_type=jax.ShapeDtypeStruct((batch, 4, N_STEPS), jnp.float32),
        mesh=plsc.VectorSubcoreMesh(core_axis_name="c", subcore_axis_name="s"),
        compiler_params=_SC_PARAMS,
        scratch_types=[
            pltpu.VMEM((rays_per_w, 16), jnp.float32),
            pltpu.VMEM((N_STEPS,), jnp.int32),
            pltpu.VMEM((N_STEPS,), jnp.int32),
            pltpu.VMEM((N_STEPS, ROW_W), jnp.int32),
            pltpu.VMEM((N_STEPS, ROW_W), jnp.int32),
            pltpu.VMEM((LANES, TR_W), jnp.int32),
            pltpu.VMEM((OUT_BATCH, 4, N_STEPS), jnp.float32),
            pltpu.SemaphoreType.DMA,
            pltpu.SemaphoreType.DMA,
        ],
    )


def _composite_body(data_ref, tri_ref, o_ref):
    d = data_ref[...]
    raw = d[:, 3, :] - 1.0
    sig = jnp.maximum(raw, 0.0) + jnp.log(1.0 + jnp.exp(-jnp.abs(raw)))
    alpha = 1.0 - jnp.exp(-STEP_SIZE * sig)
    csum_ex = lax.dot_general(
        sig, tri_ref[...], (((1,), (0,)), ((), ())),
        precision=lax.Precision.HIGHEST, preferred_element_type=jnp.float32)
    w = jnp.exp(-STEP_SIZE * csum_ex) * alpha
    outs = []
    for c in range(3):
        rgb = (1.0 + 2.0 * RGB_PADDING) / (1.0 + jnp.exp(-d[:, c, :])) \
            - RGB_PADDING
        outs.append(jnp.sum(w * rgb, axis=1, keepdims=True))
    t_last = jnp.exp(-STEP_SIZE * jnp.sum(sig, axis=1, keepdims=True))
    o_ref[...] = jnp.concatenate(outs, axis=1) + t_last * BACKGROUND


def _composite(fused, interpret=False):
    batch = fused.shape[0]
    blk = 512
    tri = (jnp.arange(N_STEPS)[:, None] < jnp.arange(N_STEPS)[None, :]
           ).astype(jnp.float32)
    return pl.pallas_call(
        _composite_body,
        grid=(batch // blk,),
        in_specs=[
            pl.BlockSpec((blk, 4, N_STEPS), lambda i: (i, 0, 0)),
            pl.BlockSpec((N_STEPS, N_STEPS), lambda i: (0, 0)),
        ],
        out_specs=pl.BlockSpec((blk, 3), lambda i: (i, 0)),
        out_shape=jax.ShapeDtypeStruct((batch, 3), jnp.float32),
        interpret=interpret,
    )(fused, tri)


def _ray_params(origins, dirs, viewdirs):
    dirs = dirs / jnp.linalg.norm(dirs, axis=-1, keepdims=True)
    viewdirs = viewdirs / jnp.linalg.norm(viewdirs, axis=-1, keepdims=True)
    invdir = 1.0 / (dirs + 1e-9)
    t1 = -origins * invdir
    t2 = t1 + invdir
    tmin = jnp.maximum(jnp.max(jnp.minimum(t1, t2), axis=-1), 0.0)
    tmax = jnp.min(jnp.maximum(t1, t2), axis=-1)
    x, y, z = viewdirs[:, 0], viewdirs[:, 1], viewdirs[:, 2]
    sh = jnp.stack([
        -SH_C1 * y,
        SH_C1 * z,
        -SH_C1 * x,
        SH_C2[0] * x * y,
        SH_C2[1] * y * z,
        SH_C2[2] * (2.0 * z * z - x * x - y * y),
        SH_C2[3] * x * z,
        SH_C2[4] * (x * x - y * y),
    ], axis=-1)
    return jnp.concatenate(
        [origins, dirs, tmin[:, None], tmax[:, None], sh], axis=1)


def kernel(origins, dirs, viewdirs, grid):
    batch = origins.shape[0]
    rp = _ray_params(origins, dirs, viewdirs)
    g_rows = grid.transpose(0, 3, 1, 2).reshape(R * DATA_DIM * R, R)
    table = _sc_permute(g_rows)
    fused = _make_sc_render(batch)(table, rp)
    return _composite(fused)
